# baseline (speedup 1.0000x reference)
import dataclasses
import functools

import jax
import jax.numpy as jnp
from jax import lax
from jax.experimental import pallas as pl
from jax.experimental.pallas import tpu as pltpu
from jax.experimental.pallas import tpu_sc as plsc

M = 16384
N = 16384
K = 16384
NNZ = 164
BATCH = 16

NUM_TILES = 32
ROWS_PER_TILE = M // NUM_TILES
CHUNK_ROWS = 16
CHUNK_IDX = CHUNK_ROWS * NNZ
NUM_CHUNKS = ROWS_PER_TILE // CHUNK_ROWS
UNROLL = 4

_COMPILER_PARAMS = pltpu.CompilerParams()
if "needs_layout_passes" in pltpu.CompilerParams.__dataclass_fields__:
    _COMPILER_PARAMS = dataclasses.replace(
        _COMPILER_PARAMS, needs_layout_passes=False)
if "use_tc_tiling_on_sc" in pltpu.CompilerParams.__dataclass_fields__:
    _COMPILER_PARAMS = dataclasses.replace(
        _COMPILER_PARAMS, use_tc_tiling_on_sc=False)


def _spmm_stage(table, cols_flat, vals_flat):
    mesh = plsc.VectorSubcoreMesh(core_axis_name="c", subcore_axis_name="s")

    @functools.partial(
        pl.kernel,
        out_type=jax.ShapeDtypeStruct((M, BATCH), jnp.float32),
        mesh=mesh,
        compiler_params=_COMPILER_PARAMS,
        scratch_types=[
            pltpu.VMEM((2, CHUNK_IDX), jnp.int32),
            pltpu.VMEM((2, CHUNK_IDX), jnp.float32),
            pltpu.VMEM((2, CHUNK_IDX, BATCH), jnp.float32),
            pltpu.VMEM((ROWS_PER_TILE, BATCH), jnp.float32),
            pltpu.SemaphoreType.DMA,
            pltpu.SemaphoreType.DMA,
            pltpu.SemaphoreType.DMA,
            pltpu.SemaphoreType.DMA,
            pltpu.SemaphoreType.DMA,
            pltpu.SemaphoreType.DMA,
        ],
    )
    def kern(table_hbm, cols_hbm, vals_hbm, out_hbm,
             cols_v, vals_v, rows_v, out_v,
             sem_c0, sem_c1, sem_v0, sem_v1, sem_g0, sem_g1):
        wid = lax.axis_index("s") * 2 + lax.axis_index("c")
        row_base = wid * ROWS_PER_TILE
        sem_c = (sem_c0, sem_c1)
        sem_v = (sem_v0, sem_v1)
        sem_g = (sem_g0, sem_g1)

        def c_copy(k, buf):
            base_idx = (row_base + k * CHUNK_ROWS) * NNZ
            return pltpu.make_async_copy(
                cols_hbm.at[pl.ds(base_idx, CHUNK_IDX)],
                cols_v.at[buf], sem_c[buf])

        def v_copy(k, buf):
            base_idx = (row_base + k * CHUNK_ROWS) * NNZ
            return pltpu.make_async_copy(
                vals_hbm.at[pl.ds(base_idx, CHUNK_IDX)],
                vals_v.at[buf], sem_v[buf])

        def gather_copies(buf):
            copies = []
            for pair in range(CHUNK_ROWS // 2):
                off = pair * (2 * NNZ)
                for (o, nn) in ((0, 128), (128, 128), (256, 72)):
                    copies.append(pltpu.make_async_copy(
                        table_hbm.at[cols_v.at[buf].at[pl.ds(off + o, nn)]],
                        rows_v.at[buf].at[pl.ds(off + o, nn)],
                        sem_g[buf]))
            return copies

        def start_gathers(buf):
            for c in gather_copies(buf):
                c.start()

        def wait_gathers(buf):
            for c in gather_copies(buf):
                c.wait()


        def compute(k, buf):
            rows_ref = rows_v.at[buf]
            vals_ref = vals_v.at[buf]

            @pl.loop(0, CHUNK_ROWS)
            def _(c):
                base = c * NNZ

                def group(b16, n, accs):
                    vblock = vals_ref[pl.ds(b16, BATCH)]
                    accs = list(accs)
                    for u in range(n):
                        row = rows_ref[b16 + u]
                        vb = vblock.at[
                            jnp.full((BATCH,), u, jnp.int32)
                        ].get(mode="promise_in_bounds")
                        accs[u % 4] = accs[u % 4] + row * vb
                    return tuple(accs)

                def body(jg, accs):
                    return group(base + jg * BATCH, BATCH, accs)

                accs = lax.fori_loop(
                    0, NNZ // BATCH, body,
                    tuple(jnp.zeros((BATCH,), jnp.float32)
                          for _ in range(4)))
                accs = group(base + (NNZ // BATCH) * BATCH,
                             NNZ % BATCH, accs)
                out_v[k * CHUNK_ROWS + c] = (
                    (accs[0] + accs[1]) + (accs[2] + accs[3]))

        c_copy(0, 0).start()
        c_copy(1, 1).start()
        v_copy(0, 0).start()
        v_copy(1, 1).start()
        c_copy(0, 0).wait()
        start_gathers(0)

        @pl.loop(0, NUM_CHUNKS // 2)
        def _(kk):
            for p in (0, 1):
                k = 2 * kk + p
                q = 1 - p

                @pl.when(k + 1 < NUM_CHUNKS)
                def _():
                    c_copy(k + 1, q).wait()
                    start_gathers(q)

                wait_gathers(p)

                @pl.when(k + 2 < NUM_CHUNKS)
                def _():
                    c_copy(k + 2, p).start()

                v_copy(k, p).wait()
                compute(k, p)

                @pl.when(k + 2 < NUM_CHUNKS)
                def _():
                    v_copy(k + 2, p).start()

        pltpu.sync_copy(out_v, out_hbm.at[pl.ds(row_base, ROWS_PER_TILE)])

    return kern(table, cols_flat, vals_flat)


def kernel(x, a_cols, a_vals, b_cols, b_vals):
    xT = x[0].T
    t = _spmm_stage(xT, b_cols.reshape(-1), b_vals.reshape(-1))
    y = _spmm_stage(t, a_cols.reshape(-1), a_vals.reshape(-1))
    return y.T[None]


# speedup vs baseline: 1.4749x; 1.4749x over previous
import dataclasses
import functools

import jax
import jax.numpy as jnp
from jax import lax
from jax.experimental import pallas as pl
from jax.experimental.pallas import tpu as pltpu
from jax.experimental.pallas import tpu_sc as plsc

M = 16384
N = 16384
K = 16384
NNZ = 164
BATCH = 16

NUM_TILES = 32
ROWS_PER_TILE = M // NUM_TILES
CHUNK_ROWS = 8
CHUNK_IDX = CHUNK_ROWS * NNZ
NUM_CHUNKS = ROWS_PER_TILE // CHUNK_ROWS
UNROLL = 4

_COMPILER_PARAMS = pltpu.CompilerParams()
if "needs_layout_passes" in pltpu.CompilerParams.__dataclass_fields__:
    _COMPILER_PARAMS = dataclasses.replace(
        _COMPILER_PARAMS, needs_layout_passes=False)
if "use_tc_tiling_on_sc" in pltpu.CompilerParams.__dataclass_fields__:
    _COMPILER_PARAMS = dataclasses.replace(
        _COMPILER_PARAMS, use_tc_tiling_on_sc=False)


def _spmm_stage(table, cols_flat, vals_flat):
    mesh = plsc.VectorSubcoreMesh(core_axis_name="c", subcore_axis_name="s")

    @functools.partial(
        pl.kernel,
        out_type=jax.ShapeDtypeStruct((M, BATCH), jnp.float32),
        mesh=mesh,
        compiler_params=_COMPILER_PARAMS,
        scratch_types=[
            pltpu.VMEM((2, CHUNK_IDX), jnp.int32),
            pltpu.VMEM((2, CHUNK_IDX), jnp.float32),
            pltpu.VMEM((2, CHUNK_IDX, BATCH), jnp.float32),
            pltpu.VMEM((ROWS_PER_TILE, BATCH), jnp.float32),
            pltpu.VMEM_SHARED((M, BATCH), jnp.float32),
            pltpu.SemaphoreType.DMA,
            pltpu.SemaphoreType.DMA,
            pltpu.SemaphoreType.DMA,
            pltpu.SemaphoreType.DMA,
            pltpu.SemaphoreType.DMA,
            pltpu.SemaphoreType.DMA,
        ],
    )
    def kern(table_hbm, cols_hbm, vals_hbm, out_hbm,
             cols_v, vals_v, rows_v, out_v, table_sh,
             sem_c0, sem_c1, sem_v0, sem_v1, sem_g0, sem_g1):
        wid = lax.axis_index("s") * 2 + lax.axis_index("c")
        row_base = wid * ROWS_PER_TILE

        sid = lax.axis_index("s")
        stripe = M // 16
        pltpu.sync_copy(table_hbm.at[pl.ds(sid * stripe, stripe)],
                        table_sh.at[pl.ds(sid * stripe, stripe)])
        plsc.subcore_barrier()
        sem_c = (sem_c0, sem_c1)
        sem_v = (sem_v0, sem_v1)
        sem_g = (sem_g0, sem_g1)

        def c_copy(k, buf):
            base_idx = (row_base + k * CHUNK_ROWS) * NNZ
            return pltpu.make_async_copy(
                cols_hbm.at[pl.ds(base_idx, CHUNK_IDX)],
                cols_v.at[buf], sem_c[buf])

        def v_copy(k, buf):
            base_idx = (row_base + k * CHUNK_ROWS) * NNZ
            return pltpu.make_async_copy(
                vals_hbm.at[pl.ds(base_idx, CHUNK_IDX)],
                vals_v.at[buf], sem_v[buf])

        def gather_copies(buf):
            copies = []
            for pair in range(CHUNK_ROWS // 2):
                off = pair * (2 * NNZ)
                for (o, nn) in ((0, 128), (128, 128), (256, 72)):
                    copies.append(pltpu.make_async_copy(
                        table_sh.at[cols_v.at[buf].at[pl.ds(off + o, nn)]],
                        rows_v.at[buf].at[pl.ds(off + o, nn)],
                        sem_g[buf]))
            return copies

        def start_gathers(buf):
            for c in gather_copies(buf):
                c.start()

        def wait_gathers(buf):
            for c in gather_copies(buf):
                c.wait()


        def compute(k, buf):
            rows_ref = rows_v.at[buf]
            vals_ref = vals_v.at[buf]

            @pl.loop(0, CHUNK_ROWS)
            def _(c):
                base = c * NNZ

                def group(b16, n, accs):
                    vblock = vals_ref[pl.ds(b16, BATCH)]
                    accs = list(accs)
                    for u in range(n):
                        row = rows_ref[b16 + u]
                        vb = vblock.at[
                            jnp.full((BATCH,), u, jnp.int32)
                        ].get(mode="promise_in_bounds")
                        accs[u % 4] = accs[u % 4] + row * vb
                    return tuple(accs)

                def body(jg, accs):
                    return group(base + jg * BATCH, BATCH, accs)

                accs = lax.fori_loop(
                    0, NNZ // BATCH, body,
                    tuple(jnp.zeros((BATCH,), jnp.float32)
                          for _ in range(4)))
                accs = group(base + (NNZ // BATCH) * BATCH,
                             NNZ % BATCH, accs)
                out_v[k * CHUNK_ROWS + c] = (
                    (accs[0] + accs[1]) + (accs[2] + accs[3]))

        c_copy(0, 0).start()
        c_copy(1, 1).start()
        v_copy(0, 0).start()
        v_copy(1, 1).start()
        c_copy(0, 0).wait()
        start_gathers(0)

        @pl.loop(0, NUM_CHUNKS // 2)
        def _(kk):
            for p in (0, 1):
                k = 2 * kk + p
                q = 1 - p

                @pl.when(k + 1 < NUM_CHUNKS)
                def _():
                    c_copy(k + 1, q).wait()
                    start_gathers(q)

                wait_gathers(p)

                @pl.when(k + 2 < NUM_CHUNKS)
                def _():
                    c_copy(k + 2, p).start()

                v_copy(k, p).wait()
                compute(k, p)

                @pl.when(k + 2 < NUM_CHUNKS)
                def _():
                    v_copy(k + 2, p).start()

        pltpu.sync_copy(out_v, out_hbm.at[pl.ds(row_base, ROWS_PER_TILE)])

    return kern(table, cols_flat, vals_flat)


def kernel(x, a_cols, a_vals, b_cols, b_vals):
    xT = x[0].T
    t = _spmm_stage(xT, b_cols.reshape(-1), b_vals.reshape(-1))
    y = _spmm_stage(t, a_cols.reshape(-1), a_vals.reshape(-1))
    return y.T[None]
